# baseline (speedup 1.0000x reference)
import jax
import jax.numpy as jnp
from jax import lax
from jax.experimental import pallas as pl

_TOP_K = 8
_BT = 1024


def _router_block(x_ref, w_ref, b_ref, ident_ref, logits_ref, weights_ref,
                  idx_ref, mask_ref):
    n_exp = w_ref.shape[0]
    bt = x_ref.shape[0]
    logits_t = lax.dot_general(w_ref[...], x_ref[...], (((1,), (1,)), ((), ())),
                               preferred_element_type=jnp.float32)
    logits_t = logits_t + b_ref[...]
    logits_ref[...] = lax.dot_general(
        logits_t, ident_ref[...], (((0,), (0,)), ((), ())),
        preferred_element_type=jnp.float32)

    erow = lax.broadcasted_iota(jnp.int32, (n_exp, bt), 0)
    bits = lax.bitcast_convert_type(logits_t, jnp.int32)
    work = bits ^ ((bits >> 31) & jnp.int32(0x7FFFFFFF))
    keys, idxs = [], []
    for _ in range(_TOP_K):
        mk = jnp.max(work, axis=0, keepdims=True)
        ik = jnp.min(jnp.where(work == mk, erow, n_exp), axis=0, keepdims=True)
        keys.append(mk)
        idxs.append(ik)
        work = jnp.where(erow == ik, jnp.int32(-2**31), work)
    kv = jnp.concatenate(keys, axis=0)
    ixt = jnp.concatenate(idxs, axis=0)
    v = lax.bitcast_convert_type(kv ^ ((kv >> 31) & jnp.int32(0x7FFFFFFF)),
                                 jnp.float32)
    e = jnp.exp(v - v[0:1, :])
    wt = e / jnp.sum(e, axis=0, keepdims=True)

    ident8 = ident_ref[0:_TOP_K, 0:_TOP_K]
    weights_ref[...] = lax.dot_general(wt, ident8, (((0,), (0,)), ((), ())),
                                       preferred_element_type=jnp.float32)
    ixf = lax.dot_general(ixt.astype(jnp.float32), ident8,
                          (((0,), (0,)), ((), ())),
                          preferred_element_type=jnp.float32)
    idx_ref[...] = ixf.astype(jnp.int32)

    e_iota = lax.broadcasted_iota(jnp.int32, (n_exp, _TOP_K, bt), 0)
    mask_ref[...] = (e_iota == ixt[None, :, :]).astype(jnp.int32)


def kernel(x, W, b):
    tokens, hidden = x.shape
    n_exp = W.shape[0]
    bt = _BT
    b2 = b.reshape(n_exp, 1)
    ident = jnp.eye(n_exp, dtype=jnp.float32)
    out_shape = (
        jax.ShapeDtypeStruct((tokens, n_exp), jnp.float32),
        jax.ShapeDtypeStruct((tokens, _TOP_K), jnp.float32),
        jax.ShapeDtypeStruct((tokens, _TOP_K), jnp.int32),
        jax.ShapeDtypeStruct((n_exp, _TOP_K, tokens), jnp.int32),
    )
    in_specs = [
        pl.BlockSpec((bt, hidden), lambda i: (i, 0)),
        pl.BlockSpec((n_exp, hidden), lambda i: (0, 0)),
        pl.BlockSpec((n_exp, 1), lambda i: (0, 0)),
        pl.BlockSpec((n_exp, n_exp), lambda i: (0, 0)),
    ]
    out_specs = (
        pl.BlockSpec((bt, n_exp), lambda i: (i, 0)),
        pl.BlockSpec((bt, _TOP_K), lambda i: (i, 0)),
        pl.BlockSpec((bt, _TOP_K), lambda i: (i, 0)),
        pl.BlockSpec((n_exp, _TOP_K, bt), lambda i: (0, 0, i)),
    )
    return pl.pallas_call(
        _router_block,
        grid=(tokens // bt,),
        in_specs=in_specs,
        out_specs=out_specs,
        out_shape=out_shape,
    )(x, W, b2, ident)


# speedup vs baseline: 1.0028x; 1.0028x over previous
import jax
import jax.numpy as jnp
from jax import lax
from jax.experimental import pallas as pl
from jax.experimental.pallas import tpu as pltpu

_TOP_K = 8
_BT = 1024


def _router_block(x_ref, w_ref, b_ref, ident_ref, logits_ref, weights_ref,
                  idx_ref, mask_ref):
    n_exp = w_ref.shape[0]
    bt = x_ref.shape[0]
    logits_t = lax.dot_general(w_ref[...], x_ref[...], (((1,), (1,)), ((), ())),
                               preferred_element_type=jnp.float32)
    logits_t = logits_t + b_ref[...]
    logits_ref[...] = lax.dot_general(
        logits_t, ident_ref[...], (((0,), (0,)), ((), ())),
        preferred_element_type=jnp.float32)

    erow = lax.broadcasted_iota(jnp.int32, (n_exp, bt), 0)
    bits = lax.bitcast_convert_type(logits_t, jnp.int32)
    work = bits ^ ((bits >> 31) & jnp.int32(0x7FFFFFFF))
    keys, idxs = [], []
    for _ in range(_TOP_K):
        mk = jnp.max(work, axis=0, keepdims=True)
        ik = jnp.min(jnp.where(work == mk, erow, n_exp), axis=0, keepdims=True)
        keys.append(mk)
        idxs.append(ik)
        work = jnp.where(erow == ik, jnp.int32(-2**31), work)
    kv = jnp.concatenate(keys, axis=0)
    ixt = jnp.concatenate(idxs, axis=0)
    v = lax.bitcast_convert_type(kv ^ ((kv >> 31) & jnp.int32(0x7FFFFFFF)),
                                 jnp.float32)
    e = jnp.exp(v - v[0:1, :])
    wt = e / jnp.sum(e, axis=0, keepdims=True)

    ident8 = ident_ref[0:_TOP_K, 0:_TOP_K]
    weights_ref[...] = lax.dot_general(wt, ident8, (((0,), (0,)), ((), ())),
                                       preferred_element_type=jnp.float32)
    ixf = lax.dot_general(ixt.astype(jnp.float32), ident8,
                          (((0,), (0,)), ((), ())),
                          preferred_element_type=jnp.float32)
    idx_ref[...] = ixf.astype(jnp.int32)

    e_iota = lax.broadcasted_iota(jnp.int32, (n_exp, _TOP_K, bt), 0)
    mask_ref[...] = (e_iota == ixt[None, :, :]).astype(jnp.int32)


def kernel(x, W, b):
    tokens, hidden = x.shape
    n_exp = W.shape[0]
    bt = _BT
    b2 = b.reshape(n_exp, 1)
    ident = jnp.eye(n_exp, dtype=jnp.float32)
    out_shape = (
        jax.ShapeDtypeStruct((tokens, n_exp), jnp.float32),
        jax.ShapeDtypeStruct((tokens, _TOP_K), jnp.float32),
        jax.ShapeDtypeStruct((tokens, _TOP_K), jnp.int32),
        jax.ShapeDtypeStruct((n_exp, _TOP_K, tokens), jnp.int32),
    )
    in_specs = [
        pl.BlockSpec((bt, hidden), lambda i: (i, 0)),
        pl.BlockSpec((n_exp, hidden), lambda i: (0, 0)),
        pl.BlockSpec((n_exp, 1), lambda i: (0, 0)),
        pl.BlockSpec((n_exp, n_exp), lambda i: (0, 0)),
    ]
    out_specs = (
        pl.BlockSpec((bt, n_exp), lambda i: (i, 0)),
        pl.BlockSpec((bt, _TOP_K), lambda i: (i, 0)),
        pl.BlockSpec((bt, _TOP_K), lambda i: (i, 0)),
        pl.BlockSpec((n_exp, _TOP_K, bt), lambda i: (0, 0, i)),
    )
    return pl.pallas_call(
        _router_block,
        grid=(tokens // bt,),
        in_specs=in_specs,
        out_specs=out_specs,
        out_shape=out_shape,
        compiler_params=pltpu.CompilerParams(
            dimension_semantics=("parallel",)),
    )(x, W, b2, ident)
